# Initial kernel scaffold; baseline (speedup 1.0000x reference)
import jax
import jax.numpy as jnp
from jax.experimental import pallas as pl


def kernel(xyz, features, W1, b1, W2, b2, W3, b3):
    raise NotImplementedError("write your pallas kernel here")


# speedup vs baseline: 25.5490x; 25.5490x over previous
import functools

import jax
import jax.numpy as jnp
from jax import lax
from jax.experimental import pallas as pl
from jax.experimental.pallas import tpu as pltpu
from jax.experimental.pallas import tpu_sc as plsc

B = 4
N = 16384
P = 1024
S = 32
CF = 16
CIN = 3 + CF
CPAD = 32
RADIUS2 = 0.2 * 0.2
NW = N // 16
NROWS = B * P
STAGE = 48


def _fps_body(xt_ref, idx_ref, nxyz_ref, dists_ref, far_ref):
    xs = xt_ref[:, 0, :]
    ys = xt_ref[:, 1, :]
    zs = xt_ref[:, 2, :]
    far_ref[...] = jnp.zeros((B, 1), jnp.int32)
    dists_ref[...] = jnp.full((B, N), 1e10, jnp.float32)
    iota = lax.broadcasted_iota(jnp.int32, (B, N), 1)

    def step(i, _):
        far = far_ref[...]
        msk = iota == far
        cx = jnp.sum(jnp.where(msk, xs, 0.0), axis=1, keepdims=True)
        cy = jnp.sum(jnp.where(msk, ys, 0.0), axis=1, keepdims=True)
        cz = jnp.sum(jnp.where(msk, zs, 0.0), axis=1, keepdims=True)
        dx = xs - cx
        dy = ys - cy
        dz = zs - cz
        d = dx * dx + dy * dy + dz * dz
        dists = jnp.minimum(dists_ref[...], d)
        dists_ref[...] = dists
        m = jnp.max(dists, axis=1, keepdims=True)
        far_new = jnp.min(jnp.where(dists == m, iota, N), axis=1,
                          keepdims=True).astype(jnp.int32)
        far_ref[...] = far_new
        idx_ref[:, pl.ds(i, 1), :] = far[:, :, None]
        cxyz = jnp.concatenate([cx, cy, cz], axis=1)
        nxyz_ref[:, pl.ds(i, 1), :] = cxyz[:, None, :]
        return 0

    lax.fori_loop(0, P, step, 0)


def _fps(xyz_t):
    return pl.pallas_call(
        _fps_body,
        out_shape=(
            jax.ShapeDtypeStruct((B, P, 1), jnp.int32),
            jax.ShapeDtypeStruct((B, P, 3), jnp.float32),
        ),
        scratch_shapes=[
            pltpu.VMEM((B, N), jnp.float32),
            pltpu.VMEM((B, 1), jnp.int32),
        ],
    )(xyz_t)


_NBLK = 4096
_PBLK = 128


def _mask_body(xyz_ref, nxt_ref, words_ref):
    xyzb = xyz_ref[0]
    nx = nxt_ref[0]
    ab = lax.dot_general(xyzb, nx, (((1,), (0,)), ((), ())),
                         preferred_element_type=jnp.float32)
    b2 = jnp.sum(xyzb * xyzb, axis=1, keepdims=True)
    a2 = jnp.sum(nx * nx, axis=0, keepdims=True)
    d2 = b2 + a2 - 2.0 * ab
    m = (d2 <= RADIUS2).astype(jnp.float32)
    m3 = m.reshape(_NBLK // 16, 16, _PBLK)
    pw = (1 << lax.broadcasted_iota(jnp.int32, (1, 16, 1), 1)).astype(
        jnp.float32)
    w = jnp.sum(m3 * pw, axis=1)
    words_ref[0] = w.astype(jnp.int32)


def _maskpack(xyz, nxt):
    return pl.pallas_call(
        _mask_body,
        grid=(B, P // _PBLK, N // _NBLK),
        in_specs=[
            pl.BlockSpec((1, _NBLK, 3), lambda b, p, n: (b, n, 0)),
            pl.BlockSpec((1, 3, _PBLK), lambda b, p, n: (b, 0, p)),
        ],
        out_specs=pl.BlockSpec((1, _NBLK // 16, _PBLK),
                               lambda b, p, n: (b, n, p)),
        out_shape=jax.ShapeDtypeStruct((B, NW, P), jnp.int32),
    )(xyz, nxt)


_RPW = NROWS // 32
_RB = 8


def _sc_body(words_hbm, pts_hbm, out_hbm, words8_v, stage_v, gidx8_v,
             gath8_v, tot_s, sem_g):
    wid = lax.axis_index("s") * 2 + lax.axis_index("c")
    base = wid * _RPW
    bofs = (base // P) * N
    lane = lax.broadcasted_iota(jnp.int32, (16,), 0)

    def do_block(blk, _):
        row0 = base + blk * _RB
        pltpu.sync_copy(words_hbm.at[pl.ds(row0, _RB)], words8_v)

        def do_row(j, _):
            tot_s[0] = 0

            def do_group(g, _):
                @pl.when(tot_s[0] < S)
                def _():
                    wv = words8_v[j, pl.ds(g * 16, 16)]
                    anyz = plsc.all_reduce_population_count(wv != 0)[0]

                    @pl.when(anyz > 0)
                    def _():
                        def do_word(i, _):
                            w = jnp.sum(jnp.where(lane == i, wv, 0))

                            @pl.when(w != 0)
                            def _():
                                bits = jnp.bitwise_and(
                                    lax.shift_right_logical(
                                        jnp.broadcast_to(w, (16,)), lane), 1)
                                m = bits == 1
                                t0 = tot_s[0]
                                ones = jnp.where(m, 1, 0).astype(jnp.int32)
                                pcs = plsc.cumsum(ones)
                                pos = t0 + pcs - 1
                                okm = jnp.logical_and(m, pos < STAGE)
                                idxv = (g * 16 + i) * 16 + lane
                                plsc.store_scatter(stage_v, [pos], idxv,
                                                   mask=okm)
                                pc = plsc.all_reduce_population_count(m)
                                tot_s[0] = t0 + pc[0]
                            return 0

                        lax.fori_loop(0, 16, do_word, 0)
                return 0

            lax.fori_loop(0, NW // 16, do_group, 0)

            t = tot_s[0]
            s0 = stage_v[pl.ds(0, 16)]
            s1 = stage_v[pl.ds(16, 16)]
            first = jnp.where(t > 0, s0[0], 0)
            v0 = jnp.where(lane < t, s0, first)
            v1 = jnp.where(lane + 16 < t, s1, first)
            gidx8_v[j, pl.ds(0, 16)] = v0 + bofs
            gidx8_v[j, pl.ds(16, 16)] = v1 + bofs
            return 0

        lax.fori_loop(0, _RB, do_row, 0)

        handles = []
        for j in range(_RB):
            handles.append(
                pltpu.async_copy(pts_hbm.at[gidx8_v.at[j]], gath8_v.at[j],
                                 sem_g))
        for h in handles:
            h.wait()
        pltpu.sync_copy(gath8_v, out_hbm.at[pl.ds(row0, _RB)])
        return 0

    lax.fori_loop(0, _RPW // _RB, do_block, 0)


@functools.lru_cache(maxsize=None)
def _sc_select_gather_fn():
    return pl.kernel(
        _sc_body,
        out_type=jax.ShapeDtypeStruct((NROWS, S, CPAD), jnp.float32),
        mesh=plsc.VectorSubcoreMesh(core_axis_name="c", subcore_axis_name="s"),
        scratch_types=[
            pltpu.VMEM((_RB, NW), jnp.int32),
            pltpu.VMEM((STAGE + 16, ), jnp.int32),
            pltpu.VMEM((_RB, S), jnp.int32),
            pltpu.VMEM((_RB, S, CPAD), jnp.float32),
            pltpu.SMEM((4,), jnp.int32),
            pltpu.SemaphoreType.DMA,
        ],
        compiler_params=pltpu.CompilerParams(needs_layout_passes=False,
                                             use_tc_tiling_on_sc=False),
    )


def _sc_select_gather(words, pts):
    return _sc_select_gather_fn()(words, pts)


_GBLK = 128


def _mlp_body(g_ref, nx_ref, w1_ref, b1_ref, w2_ref, b2_ref, w3_ref, b3_ref,
              o_ref):
    x = g_ref[...].reshape(_GBLK * S, CPAD)
    nx = nx_ref[...]
    w1 = w1_ref[...]
    t1 = jnp.dot(x, w1, preferred_element_type=jnp.float32)
    badj = b1_ref[...][None, :] - jnp.dot(nx, w1[0:3, :],
                                          preferred_element_type=jnp.float32)
    h1 = jnp.maximum(t1.reshape(_GBLK, S, 32) + badj[:, None, :], 0.0)
    h1 = h1.reshape(_GBLK * S, 32)
    h2 = jnp.maximum(
        jnp.dot(h1, w2_ref[...], preferred_element_type=jnp.float32)
        + b2_ref[...][None, :], 0.0)
    h3 = jnp.maximum(
        jnp.dot(h2, w3_ref[...], preferred_element_type=jnp.float32)
        + b3_ref[...][None, :], 0.0)
    o_ref[...] = jnp.max(h3.reshape(_GBLK, S, 64), axis=1)


def _mlp(gath, nxf, W1, b1, W2, b2, W3, b3):
    nb = NROWS // _GBLK
    return pl.pallas_call(
        _mlp_body,
        grid=(nb,),
        in_specs=[
            pl.BlockSpec((_GBLK, S, CPAD), lambda i: (i, 0, 0)),
            pl.BlockSpec((_GBLK, 3), lambda i: (i, 0)),
            pl.BlockSpec((CPAD, 32), lambda i: (0, 0)),
            pl.BlockSpec((32,), lambda i: (0,)),
            pl.BlockSpec((32, 32), lambda i: (0, 0)),
            pl.BlockSpec((32,), lambda i: (0,)),
            pl.BlockSpec((32, 64), lambda i: (0, 0)),
            pl.BlockSpec((64,), lambda i: (0,)),
        ],
        out_specs=pl.BlockSpec((_GBLK, 64), lambda i: (i, 0)),
        out_shape=jax.ShapeDtypeStruct((NROWS, 64), jnp.float32),
    )(gath, nxf, W1, b1, W2, b2, W3, b3)


def kernel(xyz, features, W1, b1, W2, b2, W3, b3):
    xyz_t = xyz.transpose(0, 2, 1)
    fps3, new_xyz = _fps(xyz_t)
    fps_idx = fps3.reshape(B, P)
    words_t = _maskpack(xyz, new_xyz.transpose(0, 2, 1))
    words = words_t.transpose(0, 2, 1).reshape(NROWS, NW)
    pts = jnp.pad(
        jnp.concatenate([xyz, features], axis=-1).reshape(B * N, CIN),
        ((0, 0), (0, CPAD - CIN)))
    gath = _sc_select_gather(words, pts)
    W1p = jnp.pad(W1, ((0, CPAD - CIN), (0, 0)))
    out = _mlp(gath, new_xyz.reshape(NROWS, 3), W1p, b1, W2, b2, W3, b3)
    new_features = out.reshape(B, P, 64).transpose(0, 2, 1)
    return new_xyz, new_features, fps_idx


# speedup vs baseline: 27.8683x; 1.0908x over previous
import functools

import jax
import jax.numpy as jnp
from jax import lax
from jax.experimental import pallas as pl
from jax.experimental.pallas import tpu as pltpu
from jax.experimental.pallas import tpu_sc as plsc

B = 4
N = 16384
P = 1024
S = 32
CF = 16
CIN = 3 + CF
CPAD = 32
RADIUS2 = 0.2 * 0.2
NW = N // 16
NROWS = B * P
STAGE = 48


def _fps_body(xt_ref, idx_ref, nxyz_ref, dists_ref, far_ref):
    xs = xt_ref[:, 0, :]
    ys = xt_ref[:, 1, :]
    zs = xt_ref[:, 2, :]
    far_ref[...] = jnp.zeros((B, 1), jnp.int32)
    dists_ref[...] = jnp.full((B, N), 1e10, jnp.float32)
    iota = lax.broadcasted_iota(jnp.int32, (B, N), 1)

    def step(i, _):
        far = far_ref[...]
        msk = iota == far
        cx = jnp.sum(jnp.where(msk, xs, 0.0), axis=1, keepdims=True)
        cy = jnp.sum(jnp.where(msk, ys, 0.0), axis=1, keepdims=True)
        cz = jnp.sum(jnp.where(msk, zs, 0.0), axis=1, keepdims=True)
        dx = xs - cx
        dy = ys - cy
        dz = zs - cz
        d = dx * dx + dy * dy + dz * dz
        dists = jnp.minimum(dists_ref[...], d)
        dists_ref[...] = dists
        m = jnp.max(dists, axis=1, keepdims=True)
        far_new = jnp.min(jnp.where(dists == m, iota, N), axis=1,
                          keepdims=True).astype(jnp.int32)
        far_ref[...] = far_new
        idx_ref[:, pl.ds(i, 1), :] = far[:, :, None]
        cxyz = jnp.concatenate([cx, cy, cz], axis=1)
        nxyz_ref[:, pl.ds(i, 1), :] = cxyz[:, None, :]
        return 0

    lax.fori_loop(0, P, step, 0)


def _fps(xyz_t):
    return pl.pallas_call(
        _fps_body,
        out_shape=(
            jax.ShapeDtypeStruct((B, P, 1), jnp.int32),
            jax.ShapeDtypeStruct((B, P, 3), jnp.float32),
        ),
        scratch_shapes=[
            pltpu.VMEM((B, N), jnp.float32),
            pltpu.VMEM((B, 1), jnp.int32),
        ],
    )(xyz_t)


_NBLK = 4096
_PBLK = 128


def _mask_body(xyz_ref, nxt_ref, words_ref):
    xyzb = xyz_ref[0]
    nx = nxt_ref[0]
    ab = lax.dot_general(xyzb, nx, (((1,), (0,)), ((), ())),
                         preferred_element_type=jnp.float32)
    b2 = jnp.sum(xyzb * xyzb, axis=1, keepdims=True)
    a2 = jnp.sum(nx * nx, axis=0, keepdims=True)
    d2 = b2 + a2 - 2.0 * ab
    m = (d2 <= RADIUS2).astype(jnp.float32)
    m3 = m.reshape(_NBLK // 16, 16, _PBLK)
    pw = (1 << lax.broadcasted_iota(jnp.int32, (1, 16, 1), 1)).astype(
        jnp.float32)
    w = jnp.sum(m3 * pw, axis=1)
    words_ref[0] = w.astype(jnp.int32)


def _maskpack(xyz, nxt):
    return pl.pallas_call(
        _mask_body,
        grid=(B, P // _PBLK, N // _NBLK),
        in_specs=[
            pl.BlockSpec((1, _NBLK, 3), lambda b, p, n: (b, n, 0)),
            pl.BlockSpec((1, 3, _PBLK), lambda b, p, n: (b, 0, p)),
        ],
        out_specs=pl.BlockSpec((1, _NBLK // 16, _PBLK),
                               lambda b, p, n: (b, n, p)),
        out_shape=jax.ShapeDtypeStruct((B, NW, P), jnp.int32),
    )(xyz, nxt)


_RPW = NROWS // 32
_RB = 8


def _sc_body(words_hbm, pts_hbm, out_hbm, words8_v, stage_v, gidx8_v,
             gath8_v, tot_s, sem_w, sem_g, sem_o):
    wid = lax.axis_index("s") * 2 + lax.axis_index("c")
    base = wid * _RPW
    bofs = (base // P) * N
    lane = lax.broadcasted_iota(jnp.int32, (16,), 0)
    nblk = _RPW // _RB
    pltpu.async_copy(words_hbm.at[pl.ds(base, _RB)], words8_v.at[0], sem_w)

    def do_block(blk, _):
        pb = blk % 2
        row0 = base + blk * _RB
        pltpu.make_async_copy(words_hbm.at[pl.ds(row0, _RB)],
                              words8_v.at[pb], sem_w).wait()

        @pl.when(blk + 1 < nblk)
        def _():
            pltpu.async_copy(words_hbm.at[pl.ds(row0 + _RB, _RB)],
                             words8_v.at[1 - pb], sem_w)

        @pl.when(blk >= 2)
        def _():
            pltpu.make_async_copy(gath8_v.at[pb],
                                  out_hbm.at[pl.ds(row0 - 2 * _RB, _RB)],
                                  sem_o).wait()

        def do_row(j, _):
            tot_s[0] = 0

            def do_group(g, _):
                @pl.when(tot_s[0] < S)
                def _():
                    wv = words8_v[pb, j, pl.ds(g * 16, 16)]
                    anyz = plsc.all_reduce_population_count(wv != 0)[0]

                    @pl.when(anyz > 0)
                    def _():
                        t0 = tot_s[0]
                        wpc = jnp.zeros((16,), jnp.int32)
                        for k in range(16):
                            wpc = wpc + jnp.bitwise_and(
                                lax.shift_right_logical(wv, k), 1)
                        csum = plsc.cumsum(wpc)
                        bse = csum - wpc
                        partial = jnp.zeros((16,), jnp.int32)
                        for k in range(16):
                            bits = jnp.bitwise_and(
                                lax.shift_right_logical(wv, k), 1)
                            m = bits == 1
                            pos = t0 + bse + partial
                            okm = jnp.logical_and(m, pos < STAGE)
                            idxv = g * 256 + lane * 16 + k
                            plsc.store_scatter(stage_v, [pos], idxv,
                                               mask=okm)
                            partial = partial + bits
                        tot_s[0] = t0 + csum[15]
                return 0

            lax.fori_loop(0, NW // 16, do_group, 0)

            t = tot_s[0]
            s0 = stage_v[pl.ds(0, 16)]
            s1 = stage_v[pl.ds(16, 16)]
            first = jnp.where(t > 0, s0[0], 0)
            v0 = jnp.where(lane < t, s0, first)
            v1 = jnp.where(lane + 16 < t, s1, first)
            gidx8_v[j, pl.ds(0, 16)] = v0 + bofs
            gidx8_v[j, pl.ds(16, 16)] = v1 + bofs
            pltpu.async_copy(pts_hbm.at[gidx8_v.at[j]], gath8_v.at[pb, j],
                             sem_g)
            return 0

        lax.fori_loop(0, _RB, do_row, 0)

        for j in range(_RB):
            pltpu.make_async_copy(pts_hbm.at[gidx8_v.at[j]],
                                  gath8_v.at[pb, j], sem_g).wait()
        pltpu.async_copy(gath8_v.at[pb], out_hbm.at[pl.ds(row0, _RB)], sem_o)
        return 0

    lax.fori_loop(0, nblk, do_block, 0)
    pltpu.make_async_copy(gath8_v.at[0],
                          out_hbm.at[pl.ds(base, _RB)], sem_o).wait()
    pltpu.make_async_copy(gath8_v.at[1],
                          out_hbm.at[pl.ds(base, _RB)], sem_o).wait()


@functools.lru_cache(maxsize=None)
def _sc_select_gather_fn():
    return pl.kernel(
        _sc_body,
        out_type=jax.ShapeDtypeStruct((NROWS, S, CPAD), jnp.float32),
        mesh=plsc.VectorSubcoreMesh(core_axis_name="c", subcore_axis_name="s"),
        scratch_types=[
            pltpu.VMEM((2, _RB, NW), jnp.int32),
            pltpu.VMEM((STAGE + 16, ), jnp.int32),
            pltpu.VMEM((_RB, S), jnp.int32),
            pltpu.VMEM((2, _RB, S, CPAD), jnp.float32),
            pltpu.SMEM((4,), jnp.int32),
            pltpu.SemaphoreType.DMA,
            pltpu.SemaphoreType.DMA,
            pltpu.SemaphoreType.DMA,
        ],
        compiler_params=pltpu.CompilerParams(needs_layout_passes=False,
                                             use_tc_tiling_on_sc=False),
    )


def _sc_select_gather(words, pts):
    return _sc_select_gather_fn()(words, pts)


_GBLK = 128


def _mlp_body(g_ref, nx_ref, w1_ref, b1_ref, w2_ref, b2_ref, w3_ref, b3_ref,
              o_ref):
    x = g_ref[...].reshape(_GBLK * S, CPAD)
    nx = nx_ref[...]
    w1 = w1_ref[...]
    t1 = jnp.dot(x, w1, preferred_element_type=jnp.float32)
    badj = b1_ref[...][None, :] - jnp.dot(nx, w1[0:3, :],
                                          preferred_element_type=jnp.float32)
    h1 = jnp.maximum(t1.reshape(_GBLK, S, 32) + badj[:, None, :], 0.0)
    h1 = h1.reshape(_GBLK * S, 32)
    h2 = jnp.maximum(
        jnp.dot(h1, w2_ref[...], preferred_element_type=jnp.float32)
        + b2_ref[...][None, :], 0.0)
    h3 = jnp.maximum(
        jnp.dot(h2, w3_ref[...], preferred_element_type=jnp.float32)
        + b3_ref[...][None, :], 0.0)
    o_ref[...] = jnp.max(h3.reshape(_GBLK, S, 64), axis=1)


def _mlp(gath, nxf, W1, b1, W2, b2, W3, b3):
    nb = NROWS // _GBLK
    return pl.pallas_call(
        _mlp_body,
        grid=(nb,),
        in_specs=[
            pl.BlockSpec((_GBLK, S, CPAD), lambda i: (i, 0, 0)),
            pl.BlockSpec((_GBLK, 3), lambda i: (i, 0)),
            pl.BlockSpec((CPAD, 32), lambda i: (0, 0)),
            pl.BlockSpec((32,), lambda i: (0,)),
            pl.BlockSpec((32, 32), lambda i: (0, 0)),
            pl.BlockSpec((32,), lambda i: (0,)),
            pl.BlockSpec((32, 64), lambda i: (0, 0)),
            pl.BlockSpec((64,), lambda i: (0,)),
        ],
        out_specs=pl.BlockSpec((_GBLK, 64), lambda i: (i, 0)),
        out_shape=jax.ShapeDtypeStruct((NROWS, 64), jnp.float32),
    )(gath, nxf, W1, b1, W2, b2, W3, b3)


def kernel(xyz, features, W1, b1, W2, b2, W3, b3):
    xyz_t = xyz.transpose(0, 2, 1)
    fps3, new_xyz = _fps(xyz_t)
    fps_idx = fps3.reshape(B, P)
    words_t = _maskpack(xyz, new_xyz.transpose(0, 2, 1))
    words = words_t.transpose(0, 2, 1).reshape(NROWS, NW)
    pts = jnp.pad(
        jnp.concatenate([xyz, features], axis=-1).reshape(B * N, CIN),
        ((0, 0), (0, CPAD - CIN)))
    gath = _sc_select_gather(words, pts)
    W1p = jnp.pad(W1, ((0, CPAD - CIN), (0, 0)))
    out = _mlp(gath, new_xyz.reshape(NROWS, 3), W1p, b1, W2, b2, W3, b3)
    new_features = out.reshape(B, P, 64).transpose(0, 2, 1)
    return new_xyz, new_features, fps_idx


# speedup vs baseline: 28.1654x; 1.0107x over previous
import functools

import jax
import jax.numpy as jnp
from jax import lax
from jax.experimental import pallas as pl
from jax.experimental.pallas import tpu as pltpu
from jax.experimental.pallas import tpu_sc as plsc

B = 4
N = 16384
P = 1024
S = 32
CF = 16
CIN = 3 + CF
CPAD = 32
RADIUS2 = 0.2 * 0.2
NW = N // 16
NROWS = B * P
STAGE = 48


def _fps_body(xt_ref, idx_ref, nxyz_ref, dists_ref, far_ref):
    xs = xt_ref[:, 0, :]
    ys = xt_ref[:, 1, :]
    zs = xt_ref[:, 2, :]
    far_ref[...] = jnp.zeros((B, 1), jnp.int32)
    dists_ref[...] = jnp.full((B, N), 1e10, jnp.float32)
    iota = lax.broadcasted_iota(jnp.int32, (B, N), 1)

    def step(i, _):
        far = far_ref[...]
        msk = iota == far
        cx = jnp.sum(jnp.where(msk, xs, 0.0), axis=1, keepdims=True)
        cy = jnp.sum(jnp.where(msk, ys, 0.0), axis=1, keepdims=True)
        cz = jnp.sum(jnp.where(msk, zs, 0.0), axis=1, keepdims=True)
        dx = xs - cx
        dy = ys - cy
        dz = zs - cz
        d = dx * dx + dy * dy + dz * dz
        dists = jnp.minimum(dists_ref[...], d)
        dists_ref[...] = dists
        m = jnp.max(dists, axis=1, keepdims=True)
        far_new = jnp.min(jnp.where(dists == m, iota, N), axis=1,
                          keepdims=True).astype(jnp.int32)
        far_ref[...] = far_new
        idx_ref[:, pl.ds(i, 1), :] = far[:, :, None]
        cxyz = jnp.concatenate([cx, cy, cz], axis=1)
        nxyz_ref[:, pl.ds(i, 1), :] = cxyz[:, None, :]
        return 0

    lax.fori_loop(0, P, step, 0)


def _fps(xyz_t):
    return pl.pallas_call(
        _fps_body,
        out_shape=(
            jax.ShapeDtypeStruct((B, P, 1), jnp.int32),
            jax.ShapeDtypeStruct((B, P, 3), jnp.float32),
        ),
        scratch_shapes=[
            pltpu.VMEM((B, N), jnp.float32),
            pltpu.VMEM((B, 1), jnp.int32),
        ],
    )(xyz_t)


_NBLK = 4096
_PBLK = 128


def _mask_body(xyz_ref, nxt_ref, words_ref):
    xyzb = xyz_ref[0]
    nx = nxt_ref[0]
    ab = lax.dot_general(xyzb, nx, (((1,), (0,)), ((), ())),
                         preferred_element_type=jnp.float32)
    b2 = jnp.sum(xyzb * xyzb, axis=1, keepdims=True)
    a2 = jnp.sum(nx * nx, axis=0, keepdims=True)
    d2 = b2 + a2 - 2.0 * ab
    m = (d2 <= RADIUS2).astype(jnp.float32)
    m3 = m.reshape(_NBLK // 16, 16, _PBLK)
    pw = (1 << lax.broadcasted_iota(jnp.int32, (1, 16, 1), 1)).astype(
        jnp.float32)
    w = jnp.sum(m3 * pw, axis=1)
    words_ref[0] = w.astype(jnp.int32).T


def _maskpack(xyz, nxt):
    return pl.pallas_call(
        _mask_body,
        grid=(B, P // _PBLK, N // _NBLK),
        in_specs=[
            pl.BlockSpec((1, _NBLK, 3), lambda b, p, n: (b, n, 0)),
            pl.BlockSpec((1, 3, _PBLK), lambda b, p, n: (b, 0, p)),
        ],
        out_specs=pl.BlockSpec((1, _PBLK, _NBLK // 16),
                               lambda b, p, n: (b, p, n)),
        out_shape=jax.ShapeDtypeStruct((B, P, NW), jnp.int32),
    )(xyz, nxt)


_RPW = NROWS // 32
_RB = 8


def _sc_body(words_hbm, pts_hbm, out_hbm, words8_v, stage_v, gidx8_v,
             gath8_v, tot_s, sem_w, sem_g, sem_o):
    wid = lax.axis_index("s") * 2 + lax.axis_index("c")
    base = wid * _RPW
    bofs = (base // P) * N
    lane = lax.broadcasted_iota(jnp.int32, (16,), 0)
    nblk = _RPW // _RB
    pltpu.async_copy(words_hbm.at[pl.ds(base, _RB)], words8_v.at[0], sem_w)

    def do_block(blk, _):
        pb = blk % 2
        row0 = base + blk * _RB
        pltpu.make_async_copy(words_hbm.at[pl.ds(row0, _RB)],
                              words8_v.at[pb], sem_w).wait()

        @pl.when(blk + 1 < nblk)
        def _():
            pltpu.async_copy(words_hbm.at[pl.ds(row0 + _RB, _RB)],
                             words8_v.at[1 - pb], sem_w)

        @pl.when(blk >= 2)
        def _():
            pltpu.make_async_copy(gath8_v.at[pb],
                                  out_hbm.at[pl.ds(row0 - 2 * _RB, _RB)],
                                  sem_o).wait()

        def do_row(j, _):
            tot_s[0] = 0

            def do_group(g, _):
                @pl.when(tot_s[0] < S)
                def _():
                    wv = words8_v[pb, j, pl.ds(g * 16, 16)]
                    anyz = plsc.all_reduce_population_count(wv != 0)[0]

                    @pl.when(anyz > 0)
                    def _():
                        t0 = tot_s[0]
                        wpc = jnp.zeros((16,), jnp.int32)
                        for k in range(16):
                            wpc = wpc + jnp.bitwise_and(
                                lax.shift_right_logical(wv, k), 1)
                        csum = plsc.cumsum(wpc)
                        bse = csum - wpc
                        partial = jnp.zeros((16,), jnp.int32)
                        for k in range(16):
                            bits = jnp.bitwise_and(
                                lax.shift_right_logical(wv, k), 1)
                            m = bits == 1
                            pos = t0 + bse + partial
                            okm = jnp.logical_and(m, pos < STAGE)
                            idxv = g * 256 + lane * 16 + k
                            plsc.store_scatter(stage_v, [pos], idxv,
                                               mask=okm)
                            partial = partial + bits
                        tot_s[0] = t0 + csum[15]
                return 0

            lax.fori_loop(0, NW // 16, do_group, 0)

            t = tot_s[0]
            s0 = stage_v[pl.ds(0, 16)]
            s1 = stage_v[pl.ds(16, 16)]
            first = jnp.where(t > 0, s0[0], 0)
            v0 = jnp.where(lane < t, s0, first)
            v1 = jnp.where(lane + 16 < t, s1, first)
            gidx8_v[j, pl.ds(0, 16)] = v0 + bofs
            gidx8_v[j, pl.ds(16, 16)] = v1 + bofs
            pltpu.async_copy(pts_hbm.at[gidx8_v.at[j]], gath8_v.at[pb, j],
                             sem_g)
            return 0

        lax.fori_loop(0, _RB, do_row, 0)

        for j in range(_RB):
            pltpu.make_async_copy(pts_hbm.at[gidx8_v.at[j]],
                                  gath8_v.at[pb, j], sem_g).wait()
        pltpu.async_copy(gath8_v.at[pb], out_hbm.at[pl.ds(row0, _RB)], sem_o)
        return 0

    lax.fori_loop(0, nblk, do_block, 0)
    pltpu.make_async_copy(gath8_v.at[0],
                          out_hbm.at[pl.ds(base, _RB)], sem_o).wait()
    pltpu.make_async_copy(gath8_v.at[1],
                          out_hbm.at[pl.ds(base, _RB)], sem_o).wait()


@functools.lru_cache(maxsize=None)
def _sc_select_gather_fn():
    return pl.kernel(
        _sc_body,
        out_type=jax.ShapeDtypeStruct((NROWS, S, CPAD), jnp.float32),
        mesh=plsc.VectorSubcoreMesh(core_axis_name="c", subcore_axis_name="s"),
        scratch_types=[
            pltpu.VMEM((2, _RB, NW), jnp.int32),
            pltpu.VMEM((STAGE + 16, ), jnp.int32),
            pltpu.VMEM((_RB, S), jnp.int32),
            pltpu.VMEM((2, _RB, S, CPAD), jnp.float32),
            pltpu.SMEM((4,), jnp.int32),
            pltpu.SemaphoreType.DMA,
            pltpu.SemaphoreType.DMA,
            pltpu.SemaphoreType.DMA,
        ],
        compiler_params=pltpu.CompilerParams(needs_layout_passes=False,
                                             use_tc_tiling_on_sc=False),
    )


def _sc_select_gather(words, pts):
    return _sc_select_gather_fn()(words, pts)


_GBLK = 128


def _mlp_body(g_ref, nx_ref, w1_ref, b1_ref, w2_ref, b2_ref, w3_ref, b3_ref,
              o_ref):
    x = g_ref[...].reshape(_GBLK * S, CPAD)
    nx = nx_ref[...]
    w1 = w1_ref[...]
    t1 = jnp.dot(x, w1, preferred_element_type=jnp.float32)
    badj = b1_ref[...][None, :] - jnp.dot(nx, w1[0:3, :],
                                          preferred_element_type=jnp.float32)
    h1 = jnp.maximum(t1.reshape(_GBLK, S, 32) + badj[:, None, :], 0.0)
    h1 = h1.reshape(_GBLK * S, 32)
    h2 = jnp.maximum(
        jnp.dot(h1, w2_ref[...], preferred_element_type=jnp.float32)
        + b2_ref[...][None, :], 0.0)
    h3 = jnp.maximum(
        jnp.dot(h2, w3_ref[...], preferred_element_type=jnp.float32)
        + b3_ref[...][None, :], 0.0)
    o_ref[...] = jnp.max(h3.reshape(_GBLK, S, 64), axis=1)


def _mlp(gath, nxf, W1, b1, W2, b2, W3, b3):
    nb = NROWS // _GBLK
    return pl.pallas_call(
        _mlp_body,
        grid=(nb,),
        in_specs=[
            pl.BlockSpec((_GBLK, S, CPAD), lambda i: (i, 0, 0)),
            pl.BlockSpec((_GBLK, 3), lambda i: (i, 0)),
            pl.BlockSpec((CPAD, 32), lambda i: (0, 0)),
            pl.BlockSpec((32,), lambda i: (0,)),
            pl.BlockSpec((32, 32), lambda i: (0, 0)),
            pl.BlockSpec((32,), lambda i: (0,)),
            pl.BlockSpec((32, 64), lambda i: (0, 0)),
            pl.BlockSpec((64,), lambda i: (0,)),
        ],
        out_specs=pl.BlockSpec((_GBLK, 64), lambda i: (i, 0)),
        out_shape=jax.ShapeDtypeStruct((NROWS, 64), jnp.float32),
    )(gath, nxf, W1, b1, W2, b2, W3, b3)


def kernel(xyz, features, W1, b1, W2, b2, W3, b3):
    xyz_t = xyz.transpose(0, 2, 1)
    fps3, new_xyz = _fps(xyz_t)
    fps_idx = fps3.reshape(B, P)
    words = _maskpack(xyz, new_xyz.transpose(0, 2, 1)).reshape(NROWS, NW)
    pts = jnp.pad(
        jnp.concatenate([xyz, features], axis=-1).reshape(B * N, CIN),
        ((0, 0), (0, CPAD - CIN)))
    gath = _sc_select_gather(words, pts)
    W1p = jnp.pad(W1, ((0, CPAD - CIN), (0, 0)))
    out = _mlp(gath, new_xyz.reshape(NROWS, 3), W1p, b1, W2, b2, W3, b3)
    new_features = out.reshape(B, P, 64).transpose(0, 2, 1)
    return new_xyz, new_features, fps_idx
